# Initial kernel scaffold; baseline (speedup 1.0000x reference)
import jax
import jax.numpy as jnp
from jax.experimental import pallas as pl


def kernel(input, table):
    raise NotImplementedError("write your pallas kernel here")


# speedup vs baseline: 1.4577x; 1.4577x over previous
import functools

import jax
import jax.numpy as jnp
from jax import lax
from jax.experimental import pallas as pl
from jax.experimental.pallas import tpu as pltpu
from jax.experimental.pallas import tpu_sc as plsc

_EMB = 32
_NC = 2
_NS = 16
_NW = _NC * _NS
_SUB = 128
_K = 8
_CHUNK = _K * _SUB


@functools.lru_cache(maxsize=None)
def _build(n, emb):
    bpw = n // _NW
    nchunk = bpw // _CHUNK
    mesh = plsc.VectorSubcoreMesh(core_axis_name="c", subcore_axis_name="s",
                                  num_cores=_NC, num_subcores=_NS)

    @functools.partial(
        pl.kernel,
        mesh=mesh,
        out_type=jax.ShapeDtypeStruct((n, emb), jnp.float32),
        scratch_types=[
            pltpu.VMEM((_K, _SUB), jnp.int32),
            pltpu.VMEM((_CHUNK, emb), jnp.float32),
            pltpu.SemaphoreType.DMA,
        ],
        compiler_params=pltpu.CompilerParams(use_tc_tiling_on_sc=False),
    )
    def gather_kernel(idx_hbm, table_hbm, out_hbm, idx_v, rows_v, sem):
        wid = lax.axis_index("s") * _NC + lax.axis_index("c")
        base = wid * bpw

        def chunk_body(i, carry):
            off = pl.multiple_of(base + i * _CHUNK, _CHUNK)
            row = pl.multiple_of(base // _SUB + i * _K, _K)
            pltpu.sync_copy(idx_hbm.at[pl.ds(row, _K)], idx_v)
            copies = [
                pltpu.async_copy(table_hbm.at[idx_v.at[j]],
                                 rows_v.at[pl.ds(j * _SUB, _SUB)], sem)
                for j in range(_K)
            ]
            for c in copies:
                c.wait()
            pltpu.sync_copy(rows_v, out_hbm.at[pl.ds(off, _CHUNK)])
            return carry

        lax.fori_loop(0, nchunk, chunk_body, 0)

    return gather_kernel


def kernel(input, table):
    n = input.size
    idx2d = input.reshape(n // _SUB, _SUB)
    out = _build(n, table.shape[1])(idx2d, table)
    return out.reshape(input.shape + (table.shape[1],))


# speedup vs baseline: 1.4863x; 1.0196x over previous
import functools

import jax
import jax.numpy as jnp
from jax import lax
from jax.experimental import pallas as pl
from jax.experimental.pallas import tpu as pltpu
from jax.experimental.pallas import tpu_sc as plsc

_EMB = 32
_NC = 2
_NS = 16
_NW = _NC * _NS
_SUB = 128
_K = 8
_CHUNK = _K * _SUB


@functools.lru_cache(maxsize=None)
def _build(n, emb):
    bpw = n // _NW
    nchunk = bpw // _CHUNK
    npair = (nchunk - 3) // 2
    mesh = plsc.VectorSubcoreMesh(core_axis_name="c", subcore_axis_name="s",
                                  num_cores=_NC, num_subcores=_NS)

    @functools.partial(
        pl.kernel,
        mesh=mesh,
        out_type=jax.ShapeDtypeStruct((n, emb), jnp.float32),
        scratch_types=[
            pltpu.VMEM((_K, _SUB), jnp.int32),
            pltpu.VMEM((_K, _SUB), jnp.int32),
            pltpu.VMEM((_CHUNK, emb), jnp.float32),
            pltpu.VMEM((_CHUNK, emb), jnp.float32),
            pltpu.SemaphoreType.DMA,
            pltpu.SemaphoreType.DMA,
            pltpu.SemaphoreType.DMA,
            pltpu.SemaphoreType.DMA,
        ],
        compiler_params=pltpu.CompilerParams(use_tc_tiling_on_sc=False),
    )
    def gather_kernel(idx_hbm, table_hbm, out_hbm, idx0, idx1, rows0, rows1,
                      gsem0, gsem1, osem0, osem1):
        wid = lax.axis_index("s") * _NC + lax.axis_index("c")
        base = wid * bpw
        brow = base // _SUB
        idx_v = (idx0, idx1)
        rows_v = (rows0, rows1)
        gsem = (gsem0, gsem1)
        osem = (osem0, osem1)

        def load_and_fire(c, s):
            row = pl.multiple_of(brow + c * _K, _K)
            pltpu.sync_copy(idx_hbm.at[pl.ds(row, _K)], idx_v[s])
            for j in range(_K):
                pltpu.async_copy(table_hbm.at[idx_v[s].at[j]],
                                 rows_v[s].at[pl.ds(j * _SUB, _SUB)], gsem[s])

        def drain_gathers(s):
            pltpu.make_async_copy(out_hbm.at[pl.ds(0, _CHUNK)],
                                  rows_v[s], gsem[s]).wait()

        def start_out(c, s):
            off = pl.multiple_of(base + c * _CHUNK, _CHUNK)
            pltpu.async_copy(rows_v[s], out_hbm.at[pl.ds(off, _CHUNK)], osem[s])

        def drain_out(s):
            pltpu.make_async_copy(rows_v[s], out_hbm.at[pl.ds(0, _CHUNK)],
                                  osem[s]).wait()

        def retire_then_fire(c, s):
            drain_gathers(s)
            start_out(c - 2, s)
            drain_out(s)
            load_and_fire(c, s)

        load_and_fire(0, 0)
        load_and_fire(1, 1)

        def pair_body(t, carry):
            c = 2 + 2 * t
            retire_then_fire(c, 0)
            retire_then_fire(c + 1, 1)
            return carry

        lax.fori_loop(0, npair, pair_body, 0)

        last = nchunk - 1
        retire_then_fire(last, 0)
        drain_gathers(1)
        start_out(last - 1, 1)
        drain_gathers(0)
        start_out(last, 0)
        drain_out(1)
        drain_out(0)

    return gather_kernel


def kernel(input, table):
    n = input.size
    idx2d = input.reshape(n // _SUB, _SUB)
    out = _build(n, table.shape[1])(idx2d, table)
    return out.reshape(input.shape + (table.shape[1],))
